# baseline (speedup 1.0000x reference)
import functools

import jax
import jax.numpy as jnp
from jax import lax
from jax.experimental import pallas as pl
from jax.experimental.pallas import tpu as pltpu
from jax.experimental.pallas import tpu_sc as plsc

VOCAB = 1000
DIM = 64
SENT = 50
B = 4096
OUT = SENT + 1
WPR = DIM // 2
VCHUNK = 512

try:
    _info = plsc.get_sparse_core_info()
    _NC, _NS, _L = _info.num_cores, _info.num_subcores, _info.num_lanes
except Exception:
    _NC, _NS, _L = 2, 16, 16

NW = _NC * _NS
BPW = B // NW

_mesh = plsc.VectorSubcoreMesh(
    core_axis_name="c", subcore_axis_name="s",
    num_cores=_NC, num_subcores=_NS,
)

_TGROUPS = [(0, range(_L)), (_L, range(_L)), (2 * _L, range(_L)),
            (SENT - _L, range(3 * _L - (SENT - _L), _L))]


@functools.partial(
    pl.kernel,
    out_type=jax.ShapeDtypeStruct((B, DIM), jnp.float32),
    mesh=_mesh,
    scratch_types=[
        pltpu.VMEM((VOCAB * WPR,), jnp.int32),
        pltpu.VMEM((VCHUNK, DIM), jnp.float32),
        pltpu.VMEM((BPW, SENT), jnp.int32),
        pltpu.VMEM((BPW, DIM), jnp.float32),
        pltpu.SemaphoreType.DMA,
    ],
    compiler_params=pltpu.CompilerParams(needs_layout_passes=False),
)
def _sc_pool(emb_hbm, x_hbm, out_hbm, table_v, stage_v, idx_v, pool_v, sem):
    w = lax.axis_index("s") * _NC + lax.axis_index("c")
    idx_cp = pltpu.async_copy(x_hbm.at[pl.ds(w * BPW, BPW)], idx_v, sem)

    for r0, nr in ((0, VCHUNK), (VCHUNK, VOCAB - VCHUNK)):
        pltpu.sync_copy(emb_hbm.at[pl.ds(r0, nr)], stage_v.at[pl.ds(0, nr)])

        def pack_row(r, carry, r0=r0):
            vs = [stage_v[r, pl.ds(k * _L, _L)] for k in range(4)]
            wbase = (r0 + r) * WPR
            for k in range(2):
                pk = plsc.bitcast(
                    plsc.pack(vs[2 * k], vs[2 * k + 1],
                              format=plsc.PackFormat.INTERLEAVED),
                    jnp.int32)
                table_v[pl.ds(wbase + k * _L, _L)] = pk
            return carry

        lax.fori_loop(0, nr, pack_row, jnp.int32(0))
    idx_cp.wait()

    def body(b, carry):
        accs = [jnp.zeros((_L,), jnp.float32) for _ in range(4)]
        for off, js in _TGROUPS:
            toks = idx_v[b, pl.ds(off, _L)]
            for j in js:
                base = toks[j] * WPR
                for k in range(2):
                    v = table_v[pl.ds(base + k * _L, _L)]
                    lo = lax.bitcast_convert_type(v << 16, jnp.float32)
                    hi = lax.bitcast_convert_type(v, jnp.float32)
                    accs[2 * k] = accs[2 * k] + lo
                    accs[2 * k + 1] = accs[2 * k + 1] + hi
        for k in range(4):
            pool_v[b, pl.ds(k * _L, _L)] = accs[k]
        return carry

    lax.fori_loop(0, BPW, body, jnp.int32(0))
    pltpu.sync_copy(pool_v, out_hbm.at[pl.ds(w * BPW, BPW)])


def _head_body(p_ref, wt_ref, b_ref, o_ref):
    logits = jnp.dot(p_ref[...], wt_ref[...],
                     preferred_element_type=jnp.float32)
    logits = logits + b_ref[...]
    m = jnp.max(logits, axis=-1, keepdims=True)
    e = jnp.exp(logits - m)
    o_ref[...] = e * (1.0 / jnp.sum(e, axis=-1, keepdims=True))


_HEAD_BLOCK = 512
_head = pl.pallas_call(
    _head_body,
    grid=(B // _HEAD_BLOCK,),
    in_specs=[
        pl.BlockSpec((_HEAD_BLOCK, DIM), lambda i: (i, 0)),
        pl.BlockSpec((DIM, OUT), lambda i: (0, 0)),
        pl.BlockSpec((1, OUT), lambda i: (0, 0)),
    ],
    out_specs=pl.BlockSpec((_HEAD_BLOCK, OUT), lambda i: (i, 0)),
    out_shape=jax.ShapeDtypeStruct((B, OUT), jnp.float32),
)


def kernel(x, emb, W, b):
    pooled_sum = _sc_pool(emb, x.astype(jnp.int32))
    wt = W.T.astype(jnp.float32) * (1.0 / SENT)
    return _head(pooled_sum, wt, b.reshape(1, OUT))


# speedup vs baseline: 1.4080x; 1.4080x over previous
import functools

import jax
import jax.numpy as jnp
from jax import lax
from jax.experimental import pallas as pl
from jax.experimental.pallas import tpu as pltpu
from jax.experimental.pallas import tpu_sc as plsc

VOCAB = 1000
DIM = 64
SENT = 50
B = 4096
OUT = SENT + 1
WPR = DIM // 2
VCHUNK = 64

try:
    _info = plsc.get_sparse_core_info()
    _NC, _NS, _L = _info.num_cores, _info.num_subcores, _info.num_lanes
except Exception:
    _NC, _NS, _L = 2, 16, 16

NW = _NC * _NS
BPW = B // NW

_mesh = plsc.VectorSubcoreMesh(
    core_axis_name="c", subcore_axis_name="s",
    num_cores=_NC, num_subcores=_NS,
)

_TGROUPS = [(0, range(_L)), (_L, range(_L)), (2 * _L, range(_L)),
            (SENT - _L, range(3 * _L - (SENT - _L), _L))]


@functools.partial(
    pl.kernel,
    out_type=jax.ShapeDtypeStruct((B, DIM), jnp.float32),
    mesh=_mesh,
    scratch_types=[
        pltpu.VMEM((VOCAB * WPR,), jnp.int32),
        pltpu.VMEM((VCHUNK, DIM), jnp.float32),
        pltpu.VMEM((VCHUNK * WPR,), jnp.int32),
        pltpu.VMEM((BPW, SENT), jnp.int32),
        pltpu.VMEM((BPW, DIM), jnp.float32),
        pltpu.VMEM_SHARED((VOCAB * WPR,), jnp.int32),
        pltpu.SemaphoreType.DMA,
    ],
    compiler_params=pltpu.CompilerParams(needs_layout_passes=False),
)
def _sc_pool(emb_hbm, x_hbm, out_hbm, table_v, stage_v, slice_v, idx_v,
             pool_v, spk_v, sem):
    s = lax.axis_index("s")
    w = s * _NC + lax.axis_index("c")
    idx_cp = pltpu.async_copy(x_hbm.at[pl.ds(w * BPW, BPW)], idx_v, sem)

    start = jnp.minimum(s * VCHUNK, VOCAB - VCHUNK)
    pltpu.sync_copy(emb_hbm.at[pl.ds(start, VCHUNK)], stage_v)

    def pack_row(r, carry):
        vs = [stage_v[r, pl.ds(k * _L, _L)] for k in range(4)]
        for k in range(2):
            pk = plsc.bitcast(
                plsc.pack(vs[2 * k], vs[2 * k + 1],
                          format=plsc.PackFormat.INTERLEAVED),
                jnp.int32)
            slice_v[pl.ds(r * WPR + k * _L, _L)] = pk
        return carry

    lax.fori_loop(0, VCHUNK, pack_row, jnp.int32(0))
    pltpu.sync_copy(slice_v, spk_v.at[pl.ds(start * WPR, VCHUNK * WPR)])
    plsc.subcore_barrier()
    pltpu.sync_copy(spk_v, table_v)
    idx_cp.wait()

    def body(b, carry):
        accs = [jnp.zeros((_L,), jnp.float32) for _ in range(4)]
        for off, js in _TGROUPS:
            toks = idx_v[b, pl.ds(off, _L)]
            for j in js:
                base = toks[j] * WPR
                for k in range(2):
                    v = table_v[pl.ds(base + k * _L, _L)]
                    lo = lax.bitcast_convert_type(v << 16, jnp.float32)
                    hi = lax.bitcast_convert_type(v, jnp.float32)
                    accs[2 * k] = accs[2 * k] + lo
                    accs[2 * k + 1] = accs[2 * k + 1] + hi
        for k in range(4):
            pool_v[b, pl.ds(k * _L, _L)] = accs[k]
        return carry

    lax.fori_loop(0, BPW, body, jnp.int32(0))
    pltpu.sync_copy(pool_v, out_hbm.at[pl.ds(w * BPW, BPW)])


def _head_body(p_ref, wt_ref, b_ref, o_ref):
    logits = jnp.dot(p_ref[...], wt_ref[...],
                     preferred_element_type=jnp.float32)
    logits = logits + b_ref[...]
    m = jnp.max(logits, axis=-1, keepdims=True)
    e = jnp.exp(logits - m)
    o_ref[...] = e * (1.0 / jnp.sum(e, axis=-1, keepdims=True))


_HEAD_BLOCK = 512
_head = pl.pallas_call(
    _head_body,
    grid=(B // _HEAD_BLOCK,),
    in_specs=[
        pl.BlockSpec((_HEAD_BLOCK, DIM), lambda i: (i, 0)),
        pl.BlockSpec((DIM, OUT), lambda i: (0, 0)),
        pl.BlockSpec((1, OUT), lambda i: (0, 0)),
    ],
    out_specs=pl.BlockSpec((_HEAD_BLOCK, OUT), lambda i: (i, 0)),
    out_shape=jax.ShapeDtypeStruct((B, OUT), jnp.float32),
)


def kernel(x, emb, W, b):
    pooled_sum = _sc_pool(emb, x.astype(jnp.int32))
    wt = W.T.astype(jnp.float32) * (1.0 / SENT)
    return _head(pooled_sum, wt, b.reshape(1, OUT))


# speedup vs baseline: 1.5074x; 1.0706x over previous
import functools

import jax
import jax.numpy as jnp
from jax import lax
from jax.experimental import pallas as pl
from jax.experimental.pallas import tpu as pltpu
from jax.experimental.pallas import tpu_sc as plsc

VOCAB = 1000
DIM = 64
SENT = 50
B = 4096
OUT = SENT + 1
WPR = DIM // 2
VCHUNK = 64

try:
    _info = plsc.get_sparse_core_info()
    _NC, _NS, _L = _info.num_cores, _info.num_subcores, _info.num_lanes
except Exception:
    _NC, _NS, _L = 2, 16, 16

NW = _NC * _NS
BPW = B // NW

_mesh = plsc.VectorSubcoreMesh(
    core_axis_name="c", subcore_axis_name="s",
    num_cores=_NC, num_subcores=_NS,
)

_TGROUPS = [(0, range(_L)), (_L, range(_L)), (2 * _L, range(_L)),
            (SENT - _L, range(3 * _L - (SENT - _L), _L))]


@functools.partial(
    pl.kernel,
    out_type=jax.ShapeDtypeStruct((B, DIM), jnp.float32),
    mesh=_mesh,
    scratch_types=[
        pltpu.VMEM((VOCAB * WPR,), jnp.int32),
        pltpu.VMEM((VCHUNK, DIM), jnp.float32),
        pltpu.VMEM((VCHUNK * WPR,), jnp.int32),
        pltpu.VMEM((BPW, SENT), jnp.int32),
        pltpu.VMEM((BPW, DIM), jnp.float32),
        pltpu.VMEM_SHARED((VOCAB * WPR,), jnp.int32),
        pltpu.SemaphoreType.DMA,
    ],
    compiler_params=pltpu.CompilerParams(needs_layout_passes=False),
)
def _sc_pool(emb_hbm, x_hbm, out_hbm, table_v, stage_v, slice_v, idx_v,
             pool_v, spk_v, sem):
    s = lax.axis_index("s")
    w = s * _NC + lax.axis_index("c")
    idx_cp = pltpu.async_copy(x_hbm.at[pl.ds(w * BPW, BPW)], idx_v, sem)

    start = jnp.minimum(s * VCHUNK, VOCAB - VCHUNK)
    pltpu.sync_copy(emb_hbm.at[pl.ds(start, VCHUNK)], stage_v)

    def pack_row(r, carry):
        vs = [stage_v[r, pl.ds(k * _L, _L)] for k in range(4)]
        for k in range(2):
            pk = plsc.bitcast(
                plsc.pack(vs[2 * k], vs[2 * k + 1],
                          format=plsc.PackFormat.INTERLEAVED),
                jnp.int32)
            slice_v[pl.ds(r * WPR + k * _L, _L)] = pk
        return carry

    lax.fori_loop(0, VCHUNK, pack_row, jnp.int32(0))
    pltpu.sync_copy(slice_v, spk_v.at[pl.ds(start * WPR, VCHUNK * WPR)])
    plsc.subcore_barrier()
    pltpu.sync_copy(spk_v, table_v)
    idx_cp.wait()

    def body(b, carry):
        accs = [jnp.zeros((_L,), jnp.float32) for _ in range(4)]
        for off, js in _TGROUPS:
            toks = idx_v[b, pl.ds(off, _L)]
            for j in js:
                base = toks[j] * WPR
                for k in range(2):
                    v = table_v[pl.ds(base + k * _L, _L)]
                    lo = lax.bitcast_convert_type(v << 16, jnp.float32)
                    hi = lax.bitcast_convert_type(v, jnp.float32)
                    accs[2 * k] = accs[2 * k] + lo
                    accs[2 * k + 1] = accs[2 * k + 1] + hi
        for k in range(4):
            pool_v[b, pl.ds(k * _L, _L)] = accs[k]
        return carry

    lax.fori_loop(0, BPW, body, jnp.int32(0))
    pltpu.sync_copy(pool_v, out_hbm.at[pl.ds(w * BPW, BPW)])


def _head_body(p_ref, wt_ref, b_ref, o_ref):
    logits = jnp.dot(p_ref[...], wt_ref[...],
                     preferred_element_type=jnp.float32)
    logits = logits + b_ref[...]
    m = jnp.max(logits, axis=-1, keepdims=True)
    e = jnp.exp(logits - m)
    o_ref[...] = e * (1.0 / jnp.sum(e, axis=-1, keepdims=True))


_head = pl.pallas_call(
    _head_body,
    out_shape=jax.ShapeDtypeStruct((B, OUT), jnp.float32),
)


def kernel(x, emb, W, b):
    pooled_sum = _sc_pool(emb, x.astype(jnp.int32))
    wt = W.T.astype(jnp.float32) * (1.0 / SENT)
    return _head(pooled_sum, wt, b.reshape(1, OUT))


# speedup vs baseline: 1.6808x; 1.1151x over previous
import functools

import jax
import jax.numpy as jnp
from jax import lax
from jax.experimental import pallas as pl
from jax.experimental.pallas import tpu as pltpu
from jax.experimental.pallas import tpu_sc as plsc

VOCAB = 1000
DIM = 64
SENT = 50
B = 4096
OUT = SENT + 1
WPR = DIM // 2
VCHUNK = 64

try:
    _info = plsc.get_sparse_core_info()
    _NC, _NS, _L = _info.num_cores, _info.num_subcores, _info.num_lanes
except Exception:
    _NC, _NS, _L = 2, 16, 16

NW = _NC * _NS
BPW = B // NW
NBG = BPW // _L

_mesh = plsc.VectorSubcoreMesh(
    core_axis_name="c", subcore_axis_name="s",
    num_cores=_NC, num_subcores=_NS,
)

_TGROUPS = [(0, range(_L)), (_L, range(_L)), (2 * _L, range(_L)),
            (3 * _L, range(SENT - 3 * _L))]


@functools.partial(
    pl.kernel,
    out_type=jax.ShapeDtypeStruct((B, DIM), jnp.float32),
    mesh=_mesh,
    scratch_types=[
        pltpu.VMEM((VOCAB * WPR,), jnp.int32),
        pltpu.VMEM((VCHUNK, DIM), jnp.float32),
        pltpu.VMEM((VCHUNK * WPR,), jnp.int32),
        pltpu.VMEM((SENT, BPW), jnp.int32),
        pltpu.VMEM((BPW * SENT + _L,), jnp.int32),
        pltpu.VMEM((BPW, DIM), jnp.float32),
        pltpu.VMEM_SHARED((VOCAB * WPR,), jnp.int32),
        pltpu.SemaphoreType.DMA,
    ],
    compiler_params=pltpu.CompilerParams(needs_layout_passes=False),
)
def _sc_pool(emb_hbm, xt_hbm, out_hbm, table_v, stage_v, slice_v, idxt_v,
             idx_v, pool_v, spk_v, sem):
    s = lax.axis_index("s")
    w = s * _NC + lax.axis_index("c")
    idx_cp = pltpu.async_copy(xt_hbm.at[:, pl.ds(w * BPW, BPW)], idxt_v, sem)

    start = jnp.minimum(s * VCHUNK, VOCAB - VCHUNK)
    pltpu.sync_copy(emb_hbm.at[pl.ds(start, VCHUNK)], stage_v)

    def pack_row(r, carry):
        vs = [stage_v[r, pl.ds(k * _L, _L)] for k in range(4)]
        for k in range(2):
            pk = plsc.bitcast(
                plsc.pack(vs[2 * k], vs[2 * k + 1],
                          format=plsc.PackFormat.INTERLEAVED),
                jnp.int32)
            slice_v[pl.ds(r * WPR + k * _L, _L)] = pk
        return carry

    lax.fori_loop(0, VCHUNK, pack_row, jnp.int32(0))
    pltpu.sync_copy(slice_v, spk_v.at[pl.ds(start * WPR, VCHUNK * WPR)])

    idx_cp.wait()
    giota = [lax.iota(jnp.int32, _L) * SENT + g * (_L * SENT)
             for g in range(NBG)]

    def tr_body(t, carry):
        for g in range(NBG):
            v = idxt_v[t, pl.ds(g * _L, _L)]
            plsc.store_scatter(idx_v, [giota[g] + t], v)
        return carry

    lax.fori_loop(0, SENT, tr_body, jnp.int32(0))

    plsc.subcore_barrier()
    pltpu.sync_copy(spk_v, table_v)

    def body(b, carry):
        bt = b * SENT
        accs = [jnp.zeros((_L,), jnp.float32) for _ in range(4)]
        for off, js in _TGROUPS:
            toks = idx_v[pl.ds(bt + off, _L)]
            for j in js:
                base = toks[j] * WPR
                for k in range(2):
                    v = table_v[pl.ds(base + k * _L, _L)]
                    lo = lax.bitcast_convert_type(v << 16, jnp.float32)
                    hi = lax.bitcast_convert_type(v, jnp.float32)
                    accs[2 * k] = accs[2 * k] + lo
                    accs[2 * k + 1] = accs[2 * k + 1] + hi
        for k in range(4):
            pool_v[b, pl.ds(k * _L, _L)] = accs[k]
        return carry

    lax.fori_loop(0, BPW, body, jnp.int32(0))
    pltpu.sync_copy(pool_v, out_hbm.at[pl.ds(w * BPW, BPW)])


def _head_body(p_ref, w_ref, b_ref, o_ref):
    logits = lax.dot_general(
        w_ref[...], p_ref[...], (((1,), (1,)), ((), ())),
        preferred_element_type=jnp.float32)
    logits = logits * (1.0 / SENT) + b_ref[...]
    m = jnp.max(logits, axis=0, keepdims=True)
    e = jnp.exp(logits - m)
    o_ref[...] = e * (1.0 / jnp.sum(e, axis=0, keepdims=True))


_head = pl.pallas_call(
    _head_body,
    out_shape=jax.ShapeDtypeStruct((OUT, B), jnp.float32),
)


def kernel(x, emb, W, b):
    pooled_sum = _sc_pool(emb, x.astype(jnp.int32).T)
    return _head(pooled_sum, W.astype(jnp.float32), b.reshape(OUT, 1)).T
